# Initial kernel scaffold; baseline (speedup 1.0000x reference)
import jax
import jax.numpy as jnp
from jax.experimental import pallas as pl


def kernel(adduct, instrument_type, adduct_table, instrument_type_table):
    raise NotImplementedError("write your pallas kernel here")


# speedup vs baseline: 1.2520x; 1.2520x over previous
import functools

import jax
import jax.numpy as jnp
from jax import lax
from jax.experimental import pallas as pl
from jax.experimental.pallas import tpu as pltpu
from jax.experimental.pallas import tpu_sc as plsc

BATCH = 16384
DIM = 64
ODIM = 2 * DIM

_info = plsc.get_sparse_core_info()
_NC = _info.num_cores
_NS = _info.num_subcores
_NW = _NC * _NS
_BPW = BATCH // _NW
_CH = 128
_NCHUNK = _BPW // _CH
_CPP = 2
_PR = _CPP * _CH
_NPASS = _NCHUNK // _CPP


def _build():
    mesh = plsc.VectorSubcoreMesh(core_axis_name="c", subcore_axis_name="s")

    @functools.partial(
        pl.kernel,
        mesh=mesh,
        out_type=jax.ShapeDtypeStruct((BATCH, ODIM), jnp.float32),
        scratch_types=[
            pltpu.VMEM((_NCHUNK, _CH), jnp.int32),
            pltpu.VMEM((_NCHUNK, _CH), jnp.int32),
            pltpu.VMEM((_PR, ODIM), jnp.float32),
            pltpu.VMEM((_PR, ODIM), jnp.float32),
            pltpu.SemaphoreType.DMA,
        ],
    )
    def k(adduct_hbm, instr_hbm, apad_hbm, ipad_hbm, out_hbm,
          aidx_v, iidx_v, a_v, b_v, sem):
        wid = lax.axis_index("s") * _NC + lax.axis_index("c")
        base = wid * _BPW
        row0 = wid * _NCHUNK
        pltpu.sync_copy(adduct_hbm.at[pl.ds(row0, _NCHUNK), :], aidx_v)
        pltpu.sync_copy(instr_hbm.at[pl.ds(row0, _NCHUNK), :], iidx_v)
        for p in range(_NPASS):
            copies = []
            for j in range(_CPP):
                c = p * _CPP + j
                copies.append(pltpu.async_copy(
                    apad_hbm.at[aidx_v.at[c]],
                    a_v.at[pl.ds(j * _CH, _CH)], sem))
                copies.append(pltpu.async_copy(
                    ipad_hbm.at[iidx_v.at[c]],
                    b_v.at[pl.ds(j * _CH, _CH)], sem))
            for cp in copies:
                cp.wait()

            def addrow(r, _):
                for k16 in range(ODIM // 16):
                    sl = pl.ds(k16 * 16, 16)
                    a_v[r, sl] = a_v[r, sl] + b_v[r, sl]
                return ()

            lax.fori_loop(0, _PR, addrow, ())
            pltpu.sync_copy(a_v, out_hbm.at[pl.ds(base + p * _PR, _PR), :])

    return k


_sc_kernel = _build()


def kernel(adduct, instrument_type, adduct_table, instrument_type_table):
    apad = jnp.pad(adduct_table, ((0, 0), (0, DIM)))
    ipad = jnp.pad(instrument_type_table, ((0, 0), (DIM, 0)))
    adduct2 = adduct.reshape(_NW * _NCHUNK, _CH)
    instr2 = instrument_type.reshape(_NW * _NCHUNK, _CH)
    return _sc_kernel(adduct2, instr2, apad, ipad)
